# baseline (speedup 1.0000x reference)
import functools

import jax
import jax.numpy as jnp
from jax import lax
from jax.experimental import pallas as pl
from jax.experimental.pallas import tpu as pltpu
from jax.experimental.pallas import tpu_sc as plsc

_BATCH = 128
_DIM = 64
_RPW = 8


def _gather_body(idx_hbm, cos_hbm, sin_hbm, cos_out, sin_out, idx_v, rows_v, sem):
    cid = lax.axis_index("c")
    sid = lax.axis_index("s")
    wid = sid * 2 + cid
    half = wid // 16
    base = pl.multiple_of((wid % 16) * _RPW, _RPW)
    pltpu.sync_copy(idx_hbm.at[pl.ds(base, _RPW)], idx_v.at[pl.ds(0, _RPW)])

    v = idx_v[...]
    lane = lax.iota(jnp.int32, 16)
    row_ids = [jnp.max(jnp.where(lane == j, v, 0)) for j in range(_RPW)]

    @pl.when(half == 0)
    def _():
        copies = [
            pltpu.make_async_copy(cos_hbm.at[row_ids[j]], rows_v.at[j], sem)
            for j in range(_RPW)
        ]
        for c in copies:
            c.start()
        for c in copies:
            c.wait()
        pltpu.sync_copy(rows_v, cos_out.at[pl.ds(base, _RPW)])

    @pl.when(half == 1)
    def _():
        copies = [
            pltpu.make_async_copy(sin_hbm.at[row_ids[j]], rows_v.at[j], sem)
            for j in range(_RPW)
        ]
        for c in copies:
            c.start()
        for c in copies:
            c.wait()
        pltpu.sync_copy(rows_v, sin_out.at[pl.ds(base, _RPW)])


@jax.jit
def kernel(position_ids, cos_cached, sin_cached):
    idx = position_ids.reshape(_BATCH)
    run = functools.partial(
        pl.kernel,
        mesh=plsc.VectorSubcoreMesh(core_axis_name="c", subcore_axis_name="s"),
        out_type=(
            jax.ShapeDtypeStruct((_BATCH, _DIM), jnp.float32),
            jax.ShapeDtypeStruct((_BATCH, _DIM), jnp.float32),
        ),
        scratch_types=[
            pltpu.VMEM((16,), jnp.int32),
            pltpu.VMEM((_RPW, _DIM), jnp.float32),
            pltpu.SemaphoreType.DMA,
        ],
        compiler_params=pltpu.CompilerParams(needs_layout_passes=False),
    )(_gather_body)
    cos, sin = run(idx, cos_cached, sin_cached)
    return (
        cos.reshape(1, 1, _BATCH, _DIM),
        sin.reshape(1, 1, _BATCH, _DIM),
    )


# speedup vs baseline: 1.0014x; 1.0014x over previous
import functools

import jax
import jax.numpy as jnp
from jax import lax
from jax.experimental import pallas as pl
from jax.experimental.pallas import tpu as pltpu
from jax.experimental.pallas import tpu_sc as plsc

_BATCH = 128
_DIM = 64
_RPW = 8


def _gather_body(idx_hbm, cos_hbm, sin_hbm, cos_out, sin_out, idx_v, rows_v, sem):
    cid = lax.axis_index("c")
    sid = lax.axis_index("s")
    wid = sid * 2 + cid
    half = wid // 16
    base = pl.multiple_of((wid % 16) * _RPW, _RPW)
    pltpu.sync_copy(idx_hbm.at[pl.ds(base, _RPW)], idx_v.at[pl.ds(0, _RPW)])

    v = idx_v[...]
    lane = lax.iota(jnp.int32, 16)
    row_ids = [jnp.max(jnp.where(lane == j, v, 0)) for j in range(_RPW)]

    @pl.when(half == 0)
    def _():
        copies = [
            pltpu.make_async_copy(cos_hbm.at[row_ids[j]], rows_v.at[j], sem)
            for j in range(_RPW)
        ]
        for c in copies:
            c.start()
        for c in copies:
            c.wait()
        pltpu.sync_copy(rows_v, cos_out.at[pl.ds(base, _RPW)])

    @pl.when(half == 1)
    def _():
        copies = [
            pltpu.make_async_copy(sin_hbm.at[row_ids[j]], rows_v.at[j], sem)
            for j in range(_RPW)
        ]
        for c in copies:
            c.start()
        for c in copies:
            c.wait()
        pltpu.sync_copy(rows_v, sin_out.at[pl.ds(base, _RPW)])


@jax.jit
def kernel(position_ids, cos_cached, sin_cached):
    idx = position_ids.reshape(_BATCH)
    run = functools.partial(
        pl.kernel,
        mesh=plsc.VectorSubcoreMesh(core_axis_name="c", subcore_axis_name="s"),
        out_type=(
            jax.ShapeDtypeStruct((_BATCH, _DIM), jnp.float32),
            jax.ShapeDtypeStruct((_BATCH, _DIM), jnp.float32),
        ),
        scratch_types=[
            pltpu.VMEM((16,), jnp.int32),
            pltpu.VMEM((_RPW, _DIM), jnp.float32),
            pltpu.SemaphoreType.DMA,
        ],
        compiler_params=pltpu.CompilerParams(
            needs_layout_passes=False, skip_device_barrier=True
        ),
    )(_gather_body)
    cos, sin = run(idx, cos_cached, sin_cached)
    return (
        cos.reshape(1, 1, _BATCH, _DIM),
        sin.reshape(1, 1, _BATCH, _DIM),
    )


# speedup vs baseline: 1.0059x; 1.0045x over previous
import functools

import jax
import jax.numpy as jnp
from jax import lax
from jax.experimental import pallas as pl
from jax.experimental.pallas import tpu as pltpu
from jax.experimental.pallas import tpu_sc as plsc

_BATCH = 128
_DIM = 64
_RPW = 8


def _gather_body(idx_hbm, cos_hbm, sin_hbm, cos_out, sin_out, idx_v, rows_v, sem):
    cid = lax.axis_index("c")
    sid = lax.axis_index("s")
    wid = sid * 2 + cid
    half = wid // 16
    base = pl.multiple_of((wid % 16) * _RPW, _RPW)
    pltpu.sync_copy(idx_hbm.at[pl.ds(base, _RPW)], idx_v.at[pl.ds(0, _RPW)])

    v = idx_v[...]
    row_ids = [v[j] for j in range(_RPW)]

    @pl.when(half == 0)
    def _():
        copies = [
            pltpu.make_async_copy(cos_hbm.at[row_ids[j]], rows_v.at[j], sem)
            for j in range(_RPW)
        ]
        for c in copies:
            c.start()
        for c in copies:
            c.wait()
        pltpu.sync_copy(rows_v, cos_out.at[pl.ds(base, _RPW)])

    @pl.when(half == 1)
    def _():
        copies = [
            pltpu.make_async_copy(sin_hbm.at[row_ids[j]], rows_v.at[j], sem)
            for j in range(_RPW)
        ]
        for c in copies:
            c.start()
        for c in copies:
            c.wait()
        pltpu.sync_copy(rows_v, sin_out.at[pl.ds(base, _RPW)])


@jax.jit
def kernel(position_ids, cos_cached, sin_cached):
    idx = position_ids.reshape(_BATCH)
    run = functools.partial(
        pl.kernel,
        mesh=plsc.VectorSubcoreMesh(core_axis_name="c", subcore_axis_name="s"),
        out_type=(
            jax.ShapeDtypeStruct((_BATCH, _DIM), jnp.float32),
            jax.ShapeDtypeStruct((_BATCH, _DIM), jnp.float32),
        ),
        scratch_types=[
            pltpu.VMEM((16,), jnp.int32),
            pltpu.VMEM((_RPW, _DIM), jnp.float32),
            pltpu.SemaphoreType.DMA,
        ],
        compiler_params=pltpu.CompilerParams(skip_device_barrier=True),
    )(_gather_body)
    cos, sin = run(idx, cos_cached, sin_cached)
    return (
        cos.reshape(1, 1, _BATCH, _DIM),
        sin.reshape(1, 1, _BATCH, _DIM),
    )


# speedup vs baseline: 1.0513x; 1.0452x over previous
import functools

import jax
import jax.numpy as jnp
from jax import lax
from jax.experimental import pallas as pl
from jax.experimental.pallas import tpu as pltpu
from jax.experimental.pallas import tpu_sc as plsc

_BATCH = 128
_DIM = 64
_HALF = 32
_RPW = 8
_LANES = 16


def _gather_body(idx_hbm, cos_hbm, sin_hbm, cos_out, sin_out, idx_v, tiles_v, rows_v, sem):
    cid = lax.axis_index("c")
    sid = lax.axis_index("s")
    base = pl.multiple_of(sid * _RPW, _RPW)
    pltpu.sync_copy(idx_hbm.at[pl.ds(base, _RPW)], idx_v.at[pl.ds(0, _RPW)])

    v = idx_v[...]
    lanes16 = lax.iota(jnp.int32, _LANES)
    vq = v >> 7
    vl = v & 127
    col_tiles = [jnp.max(jnp.where(lanes16 == j, vq, 0)) for j in range(_RPW)]
    lanes = [jnp.max(jnp.where(lanes16 == j, vl, 0)) for j in range(_RPW)]

    def gather_half(tab_hbm, out_hbm):
        copies = [
            pltpu.make_async_copy(
                tab_hbm.at[pl.ds(0, 4), col_tiles[j]], tiles_v.at[j], sem
            )
            for j in range(_RPW)
        ]
        for c in copies:
            c.start()
        for c in copies:
            c.wait()
        for j in range(_RPW):
            lane = jnp.full((_LANES,), lanes[j], jnp.int32)
            rowj = jnp.full((_LANES,), j, jnp.int32)
            for k in range(_HALF // _LANES):
                rows = lanes16 + k * _LANES
                vals = plsc.load_gather(
                    tiles_v.at[j], [rows >> 3, rows & 7, lane]
                )
                plsc.store_scatter(rows_v, [rowj, rows], vals)
                plsc.store_scatter(rows_v, [rowj, rows + _HALF], vals)
        pltpu.sync_copy(rows_v, out_hbm.at[pl.ds(base, _RPW)])

    @pl.when(cid == 0)
    def _():
        gather_half(cos_hbm, cos_out)

    @pl.when(cid == 1)
    def _():
        gather_half(sin_hbm, sin_out)


@jax.jit
def kernel(position_ids, cos_cached, sin_cached):
    idx = position_ids.reshape(_BATCH)
    run = functools.partial(
        pl.kernel,
        mesh=plsc.VectorSubcoreMesh(core_axis_name="c", subcore_axis_name="s"),
        out_type=(
            jax.ShapeDtypeStruct((_BATCH, _DIM), jnp.float32),
            jax.ShapeDtypeStruct((_BATCH, _DIM), jnp.float32),
        ),
        scratch_types=[
            pltpu.VMEM((_LANES,), jnp.int32),
            pltpu.VMEM((_RPW, 4, 8, 128), jnp.float32),
            pltpu.VMEM((_RPW, _DIM), jnp.float32),
            pltpu.SemaphoreType.DMA,
        ],
        compiler_params=pltpu.CompilerParams(
            needs_layout_passes=False, skip_device_barrier=True
        ),
    )(_gather_body)
    cos4 = cos_cached.T.reshape(8, 256, 8, 128)
    sin4 = sin_cached.T.reshape(8, 256, 8, 128)
    cos, sin = run(idx, cos4, sin4)
    return (
        cos.reshape(1, 1, _BATCH, _DIM),
        sin.reshape(1, 1, _BATCH, _DIM),
    )


# speedup vs baseline: 1.6396x; 1.5596x over previous
import functools

import jax
import jax.numpy as jnp
from jax import lax
from jax.experimental import pallas as pl
from jax.experimental.pallas import tpu as pltpu
from jax.experimental.pallas import tpu_sc as plsc

_BATCH = 128
_DIM = 64
_HALF = 32
_RPW = 8
_LANES = 16


def _gather_body(idx_hbm, cos_hbm, sin_hbm, cos_out, sin_out, idx_v, tiles_v, rows_v, sem):
    sid = lax.axis_index("s")
    base = pl.multiple_of(sid * _RPW, _RPW)
    pltpu.sync_copy(idx_hbm.at[pl.ds(base, _RPW)], idx_v.at[pl.ds(0, _RPW)])

    v = idx_v[...]
    lanes16 = lax.iota(jnp.int32, _LANES)
    vq = v >> 7
    vl = v & 127
    col_tiles = [jnp.max(jnp.where(lanes16 == j, vq, 0)) for j in range(_RPW)]
    lanes = [jnp.max(jnp.where(lanes16 == j, vl, 0)) for j in range(_RPW)]

    def gather_half(tab_hbm, out_hbm):
        copies = [
            pltpu.make_async_copy(
                tab_hbm.at[pl.ds(0, 4), col_tiles[j]], tiles_v.at[j], sem
            )
            for j in range(_RPW)
        ]
        for c in copies:
            c.start()
        for c in copies:
            c.wait()
        for j in range(_RPW):
            lane = jnp.full((_LANES,), lanes[j], jnp.int32)
            rowj = jnp.full((_LANES,), j, jnp.int32)
            for k in range(_HALF // _LANES):
                rows = lanes16 + k * _LANES
                vals = plsc.load_gather(
                    tiles_v.at[j], [rows >> 3, rows & 7, lane]
                )
                plsc.store_scatter(rows_v, [rowj, rows], vals)
                plsc.store_scatter(rows_v, [rowj, rows + _HALF], vals)
        pltpu.sync_copy(rows_v, out_hbm.at[pl.ds(base, _RPW)])

    gather_half(cos_hbm, cos_out)
    gather_half(sin_hbm, sin_out)


@jax.jit
def kernel(position_ids, cos_cached, sin_cached):
    idx = position_ids.reshape(_BATCH)
    run = functools.partial(
        pl.kernel,
        mesh=plsc.VectorSubcoreMesh(core_axis_name="c", subcore_axis_name="s"),
        out_type=(
            jax.ShapeDtypeStruct((_BATCH, _DIM), jnp.float32),
            jax.ShapeDtypeStruct((_BATCH, _DIM), jnp.float32),
        ),
        scratch_types=[
            pltpu.VMEM((_LANES,), jnp.int32),
            pltpu.VMEM((_RPW, 4, 8, 128), jnp.float32),
            pltpu.VMEM((_RPW, _DIM), jnp.float32),
            pltpu.SemaphoreType.DMA,
        ],
        compiler_params=pltpu.CompilerParams(
            needs_layout_passes=False, skip_device_barrier=True
        ),
    )(_gather_body)
    cos4 = cos_cached.T.reshape(8, 8, 256, 128).transpose(0, 2, 1, 3)
    sin4 = sin_cached.T.reshape(8, 8, 256, 128).transpose(0, 2, 1, 3)
    cos, sin = run(idx, cos4, sin4)
    return (
        cos.reshape(1, 1, _BATCH, _DIM),
        sin.reshape(1, 1, _BATCH, _DIM),
    )


# speedup vs baseline: 1.6916x; 1.0317x over previous
import functools

import jax
import jax.numpy as jnp
from jax import lax
from jax.experimental import pallas as pl
from jax.experimental.pallas import tpu as pltpu
from jax.experimental.pallas import tpu_sc as plsc

_BATCH = 128
_DIM = 64
_HALF = 32
_RPW = 8
_LANES = 16


def _gather_body(
    idx_hbm, cos_hbm, sin_hbm, cos_out, sin_out, idx_v, tiles_v, tiles2_v, rows_v, rows2_v, sem
):
    sid = lax.axis_index("s")
    base = pl.multiple_of(sid * _RPW, _RPW)
    pltpu.sync_copy(idx_hbm.at[pl.ds(base, _RPW)], idx_v.at[pl.ds(0, _RPW)])

    v = idx_v[...]
    lanes16 = lax.iota(jnp.int32, _LANES)
    vq = v >> 7
    vl = v & 127
    col_tiles = [jnp.max(jnp.where(lanes16 == j, vq, 0)) for j in range(_RPW)]
    lanes = [jnp.max(jnp.where(lanes16 == j, vl, 0)) for j in range(_RPW)]

    copies = [
        pltpu.make_async_copy(
            tab.at[pl.ds(0, 4), col_tiles[j]], tiles.at[j], sem
        )
        for tab, tiles in ((cos_hbm, tiles_v), (sin_hbm, tiles2_v))
        for j in range(_RPW)
    ]
    for c in copies:
        c.start()
    for c in copies:
        c.wait()

    for tiles, rows_ref, out_hbm in (
        (tiles_v, rows_v, cos_out),
        (tiles2_v, rows2_v, sin_out),
    ):
        for j in range(_RPW):
            lane = jnp.full((_LANES,), lanes[j], jnp.int32)
            rowj = jnp.full((_LANES,), j, jnp.int32)
            for k in range(_HALF // _LANES):
                rows = lanes16 + k * _LANES
                vals = plsc.load_gather(tiles.at[j], [rows >> 3, rows & 7, lane])
                plsc.store_scatter(rows_ref, [rowj, rows], vals)
                plsc.store_scatter(rows_ref, [rowj, rows + _HALF], vals)
        pltpu.sync_copy(rows_ref, out_hbm.at[pl.ds(base, _RPW)])


@jax.jit
def kernel(position_ids, cos_cached, sin_cached):
    idx = position_ids.reshape(_BATCH)
    run = functools.partial(
        pl.kernel,
        mesh=plsc.VectorSubcoreMesh(core_axis_name="c", subcore_axis_name="s"),
        out_type=(
            jax.ShapeDtypeStruct((_BATCH, _DIM), jnp.float32),
            jax.ShapeDtypeStruct((_BATCH, _DIM), jnp.float32),
        ),
        scratch_types=[
            pltpu.VMEM((_LANES,), jnp.int32),
            pltpu.VMEM((_RPW, 4, 8, 128), jnp.float32),
            pltpu.VMEM((_RPW, 4, 8, 128), jnp.float32),
            pltpu.VMEM((_RPW, _DIM), jnp.float32),
            pltpu.VMEM((_RPW, _DIM), jnp.float32),
            pltpu.SemaphoreType.DMA,
        ],
        compiler_params=pltpu.CompilerParams(
            needs_layout_passes=False, skip_device_barrier=True
        ),
    )(_gather_body)
    cos4 = cos_cached.T.reshape(8, 8, 256, 128).transpose(0, 2, 1, 3)
    sin4 = sin_cached.T.reshape(8, 8, 256, 128).transpose(0, 2, 1, 3)
    cos, sin = run(idx, cos4, sin4)
    return (
        cos.reshape(1, 1, _BATCH, _DIM),
        sin.reshape(1, 1, _BATCH, _DIM),
    )


# speedup vs baseline: 1.7835x; 1.0544x over previous
import functools

import jax
import jax.numpy as jnp
from jax import lax
from jax.experimental import pallas as pl
from jax.experimental.pallas import tpu as pltpu
from jax.experimental.pallas import tpu_sc as plsc

_BATCH = 128
_DIM = 64
_HALF = 32
_WIN = 8
_IPW = 4
_LANES = 16


def _gather_body(
    idx_hbm, cos_hbm, sin_hbm, cos_out, sin_out, idx_v, tiles_c, tiles_s, rows_c, rows_s, sem
):
    cid = lax.axis_index("c")
    sid = lax.axis_index("s")
    base = pl.multiple_of(sid * _WIN, _WIN)
    pltpu.sync_copy(idx_hbm.at[pl.ds(base, _WIN)], idx_v.at[pl.ds(0, _WIN)])

    v = idx_v[...]
    lanes16 = lax.iota(jnp.int32, _LANES)
    vq = v >> 7
    vl = v & 127
    slot = [lanes16 == (j + _IPW * cid) for j in range(_IPW)]
    col_tiles = [jnp.max(jnp.where(slot[j], vq, 0)) for j in range(_IPW)]
    lanes = [jnp.max(jnp.where(slot[j], vl, 0)) for j in range(_IPW)]

    copies = [
        pltpu.make_async_copy(tab.at[pl.ds(0, 4), col_tiles[j]], tiles.at[j], sem)
        for tab, tiles in ((cos_hbm, tiles_c), (sin_hbm, tiles_s))
        for j in range(_IPW)
    ]
    for c in copies:
        c.start()
    for c in copies:
        c.wait()

    for tiles, rows_ref, out_hbm in (
        (tiles_c, rows_c, cos_out),
        (tiles_s, rows_s, sin_out),
    ):
        for j in range(_IPW):
            lane = jnp.full((_LANES,), lanes[j], jnp.int32)
            rowj = jnp.full((_LANES,), j, jnp.int32)
            for k in range(_HALF // _LANES):
                rows = lanes16 + k * _LANES
                vals = plsc.load_gather(tiles.at[j], [rows >> 3, rows & 7, lane])
                plsc.store_scatter(rows_ref, [rowj, rows], vals)
                plsc.store_scatter(rows_ref, [rowj, rows + _HALF], vals)
        pltpu.sync_copy(rows_ref, out_hbm.at[sid, pl.ds(cid * _IPW, _IPW)])


@jax.jit
def kernel(position_ids, cos_cached, sin_cached):
    idx = position_ids.reshape(_BATCH)
    run = functools.partial(
        pl.kernel,
        mesh=plsc.VectorSubcoreMesh(core_axis_name="c", subcore_axis_name="s"),
        out_type=(
            jax.ShapeDtypeStruct((16, 8, 128), jnp.float32),
            jax.ShapeDtypeStruct((16, 8, 128), jnp.float32),
        ),
        scratch_types=[
            pltpu.VMEM((_LANES,), jnp.int32),
            pltpu.VMEM((_IPW, 4, 8, 128), jnp.float32),
            pltpu.VMEM((_IPW, 4, 8, 128), jnp.float32),
            pltpu.VMEM((_IPW, 128), jnp.float32),
            pltpu.VMEM((_IPW, 128), jnp.float32),
            pltpu.SemaphoreType.DMA,
        ],
        compiler_params=pltpu.CompilerParams(
            needs_layout_passes=False, skip_device_barrier=True
        ),
    )(_gather_body)
    cos4 = cos_cached.T.reshape(8, 8, 256, 128).transpose(0, 2, 1, 3)
    sin4 = sin_cached.T.reshape(8, 8, 256, 128).transpose(0, 2, 1, 3)
    cos3, sin3 = run(idx, cos4, sin4)

    def unview(o):
        return o.reshape(_BATCH, 128)[:, :_DIM].reshape(1, 1, _BATCH, _DIM)

    return unview(cos3), unview(sin3)


# speedup vs baseline: 1.8136x; 1.0169x over previous
import functools

import jax
import jax.numpy as jnp
from jax import lax
from jax.experimental import pallas as pl
from jax.experimental.pallas import tpu as pltpu
from jax.experimental.pallas import tpu_sc as plsc

_BATCH = 128
_DIM = 64
_HALF = 32
_WIN = 8
_IPW = 4
_LANES = 16


def _gather_body(
    idx_hbm, cos_hbm, sin_hbm, cos_out, sin_out, idx_v, tiles_c, tiles_s, rows_c, rows_s, sem
):
    cid = lax.axis_index("c")
    sid = lax.axis_index("s")
    base = pl.multiple_of(sid * _WIN, _WIN)
    pltpu.sync_copy(idx_hbm.at[pl.ds(base, _WIN)], idx_v.at[pl.ds(0, _WIN)])

    v = idx_v[...]
    lanes16 = lax.iota(jnp.int32, _LANES)
    vq = v >> 7
    vl = v & 127
    slot = [lanes16 == (j + _IPW * cid) for j in range(_IPW)]
    col_tiles = [jnp.max(jnp.where(slot[j], vq, 0)) for j in range(_IPW)]
    lanes = [jnp.max(jnp.where(slot[j], vl, 0)) for j in range(_IPW)]

    copies = [
        pltpu.make_async_copy(tab.at[pl.ds(0, 4), col_tiles[j]], tiles.at[j], sem)
        for tab, tiles in ((cos_hbm, tiles_c), (sin_hbm, tiles_s))
        for j in range(_IPW)
    ]
    for c in copies:
        c.start()
    for c in copies:
        c.wait()

    out_copies = []
    for tiles, rows_ref, out_hbm in (
        (tiles_c, rows_c, cos_out),
        (tiles_s, rows_s, sin_out),
    ):
        for j in range(_IPW):
            lane = jnp.full((_LANES,), lanes[j], jnp.int32)
            for k in range(_HALF // _LANES):
                rows = lanes16 + k * _LANES
                vals = plsc.load_gather(tiles.at[j], [rows >> 3, rows & 7, lane])
                rows_ref[j, pl.ds(k * _LANES, _LANES)] = vals
                rows_ref[j, pl.ds(k * _LANES + _HALF, _LANES)] = vals
        c = pltpu.make_async_copy(
            rows_ref, out_hbm.at[sid, pl.ds(cid * _IPW, _IPW)], sem
        )
        c.start()
        out_copies.append(c)
    for c in out_copies:
        c.wait()


@jax.jit
def kernel(position_ids, cos_cached, sin_cached):
    idx = position_ids.reshape(_BATCH)
    run = functools.partial(
        pl.kernel,
        mesh=plsc.VectorSubcoreMesh(core_axis_name="c", subcore_axis_name="s"),
        out_type=(
            jax.ShapeDtypeStruct((16, 8, 128), jnp.float32),
            jax.ShapeDtypeStruct((16, 8, 128), jnp.float32),
        ),
        scratch_types=[
            pltpu.VMEM((_LANES,), jnp.int32),
            pltpu.VMEM((_IPW, 4, 8, 128), jnp.float32),
            pltpu.VMEM((_IPW, 4, 8, 128), jnp.float32),
            pltpu.VMEM((_IPW, 128), jnp.float32),
            pltpu.VMEM((_IPW, 128), jnp.float32),
            pltpu.SemaphoreType.DMA,
        ],
        compiler_params=pltpu.CompilerParams(
            needs_layout_passes=False, skip_device_barrier=True
        ),
    )(_gather_body)
    cos4 = cos_cached.T.reshape(8, 8, 256, 128).transpose(0, 2, 1, 3)
    sin4 = sin_cached.T.reshape(8, 8, 256, 128).transpose(0, 2, 1, 3)
    cos3, sin3 = run(idx, cos4, sin4)

    def unview(o):
        return o.reshape(_BATCH, 128)[:, :_DIM].reshape(1, 1, _BATCH, _DIM)

    return unview(cos3), unview(sin3)
